# baseline (speedup 1.0000x reference)
import functools

import jax
import jax.numpy as jnp
from jax import lax
from jax.experimental import pallas as pl
from jax.experimental.pallas import tpu as pltpu
from jax.experimental.pallas import tpu_sc as plsc

VOCAB = 100000
EMB = 128
BATCH = 4096
SEQ = 50
SEQP = 56

_NC = 2
_NS = 16
_NW = _NC * _NS
_ROWS_W = BATCH // _NW
_NBUF = 8
_NGROUP = _ROWS_W // _NBUF


def _emb_body(x_hbm, table_hbm, out_hbm, idx_v, rows_v, gsem, osem):
    wid = lax.axis_index("s") * _NC + lax.axis_index("c")
    row0 = wid * _ROWS_W
    pltpu.sync_copy(x_hbm.at[wid], idx_v)

    def start_gather(j, b):
        pltpu.async_copy(table_hbm.at[idx_v.at[j]], rows_v.at[b], gsem.at[b])

    def wait_gather(b):
        pltpu.make_async_copy(
            table_hbm.at[pl.ds(0, SEQP)], rows_v.at[b], gsem.at[b]
        ).wait()

    def start_out(j, b):
        pltpu.async_copy(
            rows_v.at[b, pl.ds(0, SEQ)], out_hbm.at[row0 + j], osem.at[b]
        )

    def wait_out(b):
        pltpu.make_async_copy(
            rows_v.at[b, pl.ds(0, SEQ)], out_hbm.at[0], osem.at[b]
        ).wait()

    for b in range(_NBUF):
        start_gather(b, b)
    for b in range(_NBUF):
        wait_gather(b)
        start_out(b, b)

    def group(g, carry):
        j0 = g * _NBUF
        for b in range(_NBUF):
            wait_out(b)
            start_gather(j0 + b, b)
        for b in range(_NBUF):
            wait_gather(b)
            start_out(j0 + b, b)
        return carry

    lax.fori_loop(1, _NGROUP, group, 0)

    for b in range(_NBUF):
        wait_out(b)


_mesh = plsc.VectorSubcoreMesh(core_axis_name="c", subcore_axis_name="s")

_emb_kernel = functools.partial(
    pl.kernel,
    mesh=_mesh,
    out_type=jax.ShapeDtypeStruct((BATCH, SEQ, EMB), jnp.float32),
    scratch_types=[
        pltpu.VMEM((_ROWS_W, SEQP), jnp.int32),
        pltpu.VMEM((_NBUF, SEQP, EMB), jnp.float32),
        pltpu.SemaphoreType.DMA((_NBUF,)),
        pltpu.SemaphoreType.DMA((_NBUF,)),
    ],
)(_emb_body)


def kernel(x, table):
    xi = x.astype(jnp.int32)
    xp = jnp.pad(xi, ((0, 0), (0, SEQP - SEQ))).reshape(_NW, _ROWS_W, SEQP)
    return _emb_kernel(xp, table)


# speedup vs baseline: 7.0899x; 7.0899x over previous
import functools

import jax
import jax.numpy as jnp
from jax import lax
from jax.experimental import pallas as pl
from jax.experimental.pallas import tpu as pltpu
from jax.experimental.pallas import tpu_sc as plsc

VOCAB = 100000
EMB = 128
BATCH = 4096
SEQ = 50
SEQP = 64

_NC = 2
_NS = 16
_NW = _NC * _NS
_ROWS_W = BATCH // _NW
_CROWS = 2
_CTOK = _CROWS * SEQP
_NCHUNK = _ROWS_W // _CROWS
_NBUF = 5


def _emb_body(x_hbm, table_hbm, out_hbm, idx_v, rows_v, gsem, osem):
    wid = lax.axis_index("s") * _NC + lax.axis_index("c")
    row0 = wid * _ROWS_W
    pltpu.sync_copy(x_hbm.at[wid], idx_v)

    def start_gather(j, b):
        pltpu.async_copy(table_hbm.at[idx_v.at[j]], rows_v.at[b], gsem.at[b])

    def wait_gather(b):
        pltpu.make_async_copy(
            table_hbm.at[pl.ds(0, _CTOK)], rows_v.at[b], gsem.at[b]
        ).wait()

    def _out_src(b):
        return rows_v.at[b].reshape(_CROWS, SEQP, EMB).at[:, pl.ds(0, SEQ)]

    def start_out(j, b):
        pltpu.async_copy(
            _out_src(b), out_hbm.at[pl.ds(row0 + j * _CROWS, _CROWS)], osem.at[b]
        )

    def wait_out(b):
        pltpu.make_async_copy(
            _out_src(b), out_hbm.at[pl.ds(0, _CROWS)], osem.at[b]
        ).wait()

    for b in range(_NBUF):
        start_gather(b, b)

    ngroup = _NCHUNK // _NBUF
    rem = _NCHUNK - ngroup * _NBUF

    def group(g, carry):
        j0 = g * _NBUF
        for b in range(_NBUF):
            wait_gather(b)
            start_out(j0 + b, b)
            nxt = j0 + b + _NBUF

            @pl.when(nxt < _NCHUNK)
            def _():
                wait_out(b)
                start_gather(nxt, b)

        return carry

    lax.fori_loop(0, ngroup, group, 0)

    for b in range(rem):
        wait_gather(b)
        start_out(ngroup * _NBUF + b, b)

    for b in range(_NBUF):
        wait_out(b)


_mesh = plsc.VectorSubcoreMesh(core_axis_name="c", subcore_axis_name="s")

_emb_kernel = functools.partial(
    pl.kernel,
    mesh=_mesh,
    out_type=jax.ShapeDtypeStruct((BATCH, SEQ, EMB), jnp.float32),
    scratch_types=[
        pltpu.VMEM((_NCHUNK, _CTOK), jnp.int32),
        pltpu.VMEM((_NBUF, _CTOK, EMB), jnp.float32),
        pltpu.SemaphoreType.DMA((_NBUF,)),
        pltpu.SemaphoreType.DMA((_NBUF,)),
    ],
)(_emb_body)


def kernel(x, table):
    xi = x.astype(jnp.int32)
    xp = jnp.concatenate([xi, xi[:, : SEQP - SEQ]], axis=1)
    return _emb_kernel(xp.reshape(_NW, _NCHUNK, _CTOK), table)


# speedup vs baseline: 7.0951x; 1.0007x over previous
import functools

import jax
import jax.numpy as jnp
from jax import lax
from jax.experimental import pallas as pl
from jax.experimental.pallas import tpu as pltpu
from jax.experimental.pallas import tpu_sc as plsc

VOCAB = 100000
EMB = 128
BATCH = 4096
SEQ = 50
SEQP = 64

_NC = 2
_NS = 16
_NW = _NC * _NS
_ROWS_W = BATCH // _NW
_CROWS = 2
_CTOK = _CROWS * SEQP
_NCHUNK = _ROWS_W // _CROWS
_NBUF = 5


def _emb_body(x_hbm, table_hbm, out_hbm, idx_v, rows_v, gsem, osem):
    wid = lax.axis_index("s") * _NC + lax.axis_index("c")
    row0 = wid * _ROWS_W
    pltpu.sync_copy(x_hbm.at[wid], idx_v)

    def start_gather(j, b):
        pltpu.async_copy(table_hbm.at[idx_v.at[j]], rows_v.at[b], gsem.at[b])

    def wait_gather(b):
        pltpu.make_async_copy(
            table_hbm.at[pl.ds(0, _CTOK)], rows_v.at[b], gsem.at[b]
        ).wait()

    def _out_src(b):
        return rows_v.at[b].reshape(_CROWS, SEQP, EMB).at[:, pl.ds(0, SEQ)]

    def start_out(j, b):
        pltpu.async_copy(
            _out_src(b), out_hbm.at[pl.ds(row0 + j * _CROWS, _CROWS)], osem.at[b]
        )

    def wait_out(b):
        pltpu.make_async_copy(
            _out_src(b), out_hbm.at[pl.ds(0, _CROWS)], osem.at[b]
        ).wait()

    for b in range(_NBUF):
        start_gather(b, b)

    ngroup = _NCHUNK // _NBUF
    rem = _NCHUNK - ngroup * _NBUF

    def group(g, carry):
        j0 = g * _NBUF
        for b in range(_NBUF):
            wait_gather(b)
            start_out(j0 + b, b)
            nxt = j0 + b + _NBUF

            @pl.when(nxt < _NCHUNK)
            def _():
                wait_out(b)
                start_gather(nxt, b)

        return carry

    lax.fori_loop(0, ngroup, group, 0)

    for b in range(rem):
        wait_gather(b)
        start_out(ngroup * _NBUF + b, b)

    for b in range(_NBUF):
        wait_out(b)


_mesh = plsc.VectorSubcoreMesh(core_axis_name="c", subcore_axis_name="s")

_emb_kernel = functools.partial(
    pl.kernel,
    mesh=_mesh,
    out_type=jax.ShapeDtypeStruct((BATCH, SEQ, EMB), jnp.float32),
    compiler_params=pltpu.CompilerParams(use_tc_tiling_on_sc=True),
    scratch_types=[
        pltpu.VMEM((_NCHUNK, _CTOK), jnp.int32),
        pltpu.VMEM((_NBUF, _CTOK, EMB), jnp.float32),
        pltpu.SemaphoreType.DMA((_NBUF,)),
        pltpu.SemaphoreType.DMA((_NBUF,)),
    ],
)(_emb_body)


def kernel(x, table):
    xi = x.astype(jnp.int32)
    xp = jnp.concatenate([xi, xi[:, : SEQP - SEQ]], axis=1)
    return _emb_kernel(xp.reshape(_NW, _NCHUNK, _CTOK), table)


# speedup vs baseline: 13.7241x; 1.9343x over previous
import functools

import jax
import jax.numpy as jnp
from jax import lax
from jax.experimental import pallas as pl
from jax.experimental.pallas import tpu as pltpu
from jax.experimental.pallas import tpu_sc as plsc

VOCAB = 100000
EMB = 128
BATCH = 4096
SEQ = 50

_NC = 2
_NS = 16
_NW = _NC * _NS
_BBLK = BATCH // _NW
_NBUF = 7


def _emb_body(xt_hbm, table_hbm, out_hbm, idx_v, rows_v, gsem, osem):
    wid = lax.axis_index("s") * _NC + lax.axis_index("c")
    pltpu.sync_copy(xt_hbm.at[:, wid], idx_v)

    def start_gather(j, b):
        pltpu.async_copy(table_hbm.at[idx_v.at[j]], rows_v.at[b], gsem.at[b])

    def wait_gather(b):
        pltpu.make_async_copy(
            table_hbm.at[pl.ds(0, _BBLK)], rows_v.at[b], gsem.at[b]
        ).wait()

    def start_out(j, b):
        pltpu.async_copy(rows_v.at[b], out_hbm.at[j, wid], osem.at[b])

    def wait_out(b):
        pltpu.make_async_copy(rows_v.at[b], out_hbm.at[0, 0], osem.at[b]).wait()

    for b in range(_NBUF):
        start_gather(b, b)

    ngroup = SEQ // _NBUF
    rem = SEQ - ngroup * _NBUF

    def group(g, carry):
        j0 = g * _NBUF
        for b in range(_NBUF):
            wait_gather(b)
            start_out(j0 + b, b)
            nxt = j0 + b + _NBUF

            @pl.when(nxt < SEQ)
            def _():
                wait_out(b)
                start_gather(nxt, b)

        return carry

    lax.fori_loop(0, ngroup, group, 0)

    for b in range(rem):
        wait_gather(b)
        start_out(ngroup * _NBUF + b, b)

    for b in range(_NBUF):
        wait_out(b)


_mesh = plsc.VectorSubcoreMesh(core_axis_name="c", subcore_axis_name="s")

_emb_kernel = functools.partial(
    pl.kernel,
    mesh=_mesh,
    out_type=jax.ShapeDtypeStruct((SEQ, _NW, _BBLK, EMB), jnp.float32),
    scratch_types=[
        pltpu.VMEM((SEQ, _BBLK), jnp.int32),
        pltpu.VMEM((_NBUF, _BBLK, EMB), jnp.float32),
        pltpu.SemaphoreType.DMA((_NBUF,)),
        pltpu.SemaphoreType.DMA((_NBUF,)),
    ],
)(_emb_body)


def kernel(x, table):
    xt = x.astype(jnp.int32).T.reshape(SEQ, _NW, _BBLK)
    out = _emb_kernel(xt, table)
    return jnp.transpose(out.reshape(SEQ, BATCH, EMB), (1, 0, 2))


# speedup vs baseline: 13.7417x; 1.0013x over previous
import functools

import jax
import jax.numpy as jnp
from jax import lax
from jax.experimental import pallas as pl
from jax.experimental.pallas import tpu as pltpu
from jax.experimental.pallas import tpu_sc as plsc

VOCAB = 100000
EMB = 128
BATCH = 4096
SEQ = 50

_NC = 2
_NS = 16
_NW = _NC * _NS
_BBLK = BATCH // _NW
_NBUF = 7


def _emb_body(xt_hbm, table_hbm, out_hbm, idx_v, rows_v, gsem, osem):
    wid = lax.axis_index("s") * _NC + lax.axis_index("c")

    pltpu.sync_copy(xt_hbm.at[:, wid], idx_v)
    plsc.subcore_barrier()

    def start_gather(j, b):
        pltpu.async_copy(table_hbm.at[idx_v.at[j]], rows_v.at[b], gsem.at[b])

    def wait_gather(b):
        pltpu.make_async_copy(
            table_hbm.at[pl.ds(0, _BBLK)], rows_v.at[b], gsem.at[b]
        ).wait()

    def start_out(j, b):
        pltpu.async_copy(rows_v.at[b], out_hbm.at[j, wid], osem.at[b])

    def wait_out(b):
        pltpu.make_async_copy(rows_v.at[b], out_hbm.at[0, 0], osem.at[b]).wait()

    for b in range(_NBUF):
        start_gather(b, b)

    ngroup = SEQ // _NBUF
    rem = SEQ - ngroup * _NBUF

    def group(g, carry):
        j0 = g * _NBUF
        for b in range(_NBUF):
            wait_gather(b)
            start_out(j0 + b, b)
            nxt = j0 + b + _NBUF

            @pl.when(nxt < SEQ)
            def _():
                wait_out(b)
                start_gather(nxt, b)

        return carry

    lax.fori_loop(0, ngroup, group, 0)

    for b in range(rem):
        wait_gather(b)
        start_out(ngroup * _NBUF + b, b)

    for b in range(_NBUF):
        wait_out(b)


_mesh = plsc.VectorSubcoreMesh(core_axis_name="c", subcore_axis_name="s")

_emb_kernel = functools.partial(
    pl.kernel,
    mesh=_mesh,
    out_type=jax.ShapeDtypeStruct((SEQ, _NW, _BBLK, EMB), jnp.float32),
    scratch_types=[
        pltpu.VMEM((SEQ, _BBLK), jnp.int32),
        pltpu.VMEM((_NBUF, _BBLK, EMB), jnp.float32),
        pltpu.SemaphoreType.DMA((_NBUF,)),
        pltpu.SemaphoreType.DMA((_NBUF,)),
    ],
)(_emb_body)


def kernel(x, table):
    xt = x.astype(jnp.int32).T.reshape(SEQ, _NW, _BBLK)
    out = _emb_kernel(xt, table)
    return jnp.transpose(out.reshape(SEQ, BATCH, EMB), (1, 0, 2))
